# baseline (speedup 1.0000x reference)
import functools

import jax
import jax.numpy as jnp
from jax import lax
from jax.experimental import pallas as pl
from jax.experimental.pallas import tpu as pltpu
from jax.experimental.pallas import tpu_sc as plsc

_NC = 2
_NS = 16
_NW = _NC * _NS
_L = 16
_BB = 128


def _embed_body(seq, embed, idx_hbm, table_hbm, pos_hbm, out_hbm,
                idx_v, pos_v, rows_v, obuf_v, gs0, gs1, os0, os1):
    gsem = (gs0, gs1)
    osem = (os0, os1)
    nvec = embed // _L
    wid = lax.axis_index("s") * _NC + lax.axis_index("c")

    pltpu.sync_copy(idx_hbm.at[:, pl.ds(wid * _BB, _BB)], idx_v)
    pltpu.sync_copy(pos_hbm.at[pl.ds(0, seq)], pos_v)

    iota = lax.iota(jnp.int32, _L)
    sc_idx = []
    for c in range(nvec):
        e = iota + _L * c
        sc_idx.append((e // 8, e % 8))

    def fire(s, b):
        pltpu.async_copy(table_hbm.at[idx_v.at[s]], rows_v.at[b], gsem[b])

    def process(s, b, wait_out):
        pltpu.make_async_copy(
            table_hbm.at[pl.ds(0, _BB)], rows_v.at[b], gsem[b]).wait()
        if wait_out:
            pltpu.make_async_copy(
                obuf_v.at[b], out_hbm.at[0, :, pl.ds(0, 8)], osem[b]).wait()
        p = [pos_v[s, pl.ds(_L * c, _L)] for c in range(nvec)]

        def tok_body(t, carry):
            tvec = iota * 0 + t
            for c in range(nvec):
                g = rows_v[b, t, pl.ds(_L * c, _L)]
                plsc.store_scatter(
                    obuf_v.at[b], [sc_idx[c][0], sc_idx[c][1], tvec], g + p[c])
            return carry

        lax.fori_loop(0, _BB, tok_body, 0, unroll=2)
        pltpu.async_copy(
            obuf_v.at[b], out_hbm.at[s, :, pl.ds(wid * 8, 8)], osem[b])

    fire(0, 0)
    fire(1, 1)
    process(0, 0, False)
    fire(2, 0)
    process(1, 1, False)
    fire(3, 1)

    def loop_body(j, carry):
        for b in range(2):
            k = 2 * j + 2 + b
            process(k, b, True)
            fire(k + 2, b)
        return carry

    lax.fori_loop(0, (seq - 4) // 2, loop_body, 0)

    process(seq - 2, 0, True)
    process(seq - 1, 1, True)
    pltpu.make_async_copy(
        obuf_v.at[0], out_hbm.at[0, :, pl.ds(0, 8)], os0).wait()
    pltpu.make_async_copy(
        obuf_v.at[1], out_hbm.at[0, :, pl.ds(0, 8)], os1).wait()


def kernel(token_ids, text_table, pos_table):
    batch, seq = token_ids.shape
    vocab, embed = text_table.shape
    tok_t = token_ids.T.astype(jnp.int32)

    mesh = plsc.VectorSubcoreMesh(core_axis_name="c", subcore_axis_name="s")
    body = functools.partial(_embed_body, seq, embed)
    out4 = pl.kernel(
        body,
        out_type=jax.ShapeDtypeStruct(
            (seq, embed // 8, (batch // _BB) * 8, _BB), jnp.float32),
        mesh=mesh,
        scratch_types=[
            pltpu.VMEM((seq, _BB), jnp.int32),
            pltpu.VMEM((seq, embed), jnp.float32),
            pltpu.VMEM((2, _BB, embed), jnp.float32),
            pltpu.VMEM((2, embed // 8, 8, _BB), jnp.float32),
            pltpu.SemaphoreType.DMA,
            pltpu.SemaphoreType.DMA,
            pltpu.SemaphoreType.DMA,
            pltpu.SemaphoreType.DMA,
        ],
        compiler_params=pltpu.CompilerParams(
            use_tc_tiling_on_sc=False, needs_layout_passes=False),
        name="sc_embed_lookup",
    )(tok_t, text_table, pos_table)
    out = out4.reshape(seq, embed // 8, batch // _BB, 8, _BB)
    return out.transpose(2, 4, 0, 1, 3).reshape(batch, seq, embed)


# speedup vs baseline: 1.2362x; 1.2362x over previous
import functools

import jax
import jax.numpy as jnp
from jax import lax
from jax.experimental import pallas as pl
from jax.experimental.pallas import tpu as pltpu
from jax.experimental.pallas import tpu_sc as plsc

_NC = 2
_NS = 16
_NW = _NC * _NS
_LANES = 16

_SEG = 100
_SEGS_PER_CHUNK = 4


def _embed_body(seq, embed, n_flat, idx_hbm, table_hbm, pos_hbm, out_hbm,
                idx_v, rows_v, pos_v, gsem):
    chunk = _SEG * _SEGS_PER_CHUNK
    rows_per_chunk = chunk // seq
    per_worker = n_flat // _NW
    n_chunks = per_worker // chunk

    wid = lax.axis_index("s") * _NC + lax.axis_index("c")
    seg_base = wid * (per_worker // _SEG)
    tok_base = wid * per_worker

    pltpu.sync_copy(pos_hbm.at[pl.ds(0, seq)], pos_v)

    def chunk_body(g, carry):
        seg0 = seg_base + g * _SEGS_PER_CHUNK
        pltpu.sync_copy(idx_hbm.at[pl.ds(seg0, _SEGS_PER_CHUNK)], idx_v)
        copies = [
            pltpu.async_copy(
                table_hbm.at[idx_v.at[j]],
                rows_v.at[pl.ds(j * _SEG, _SEG)],
                gsem,
            )
            for j in range(_SEGS_PER_CHUNK)
        ]
        for cp in copies:
            cp.wait()

        def add_body(r, c2):
            for c in range(embed // _LANES):
                p = pos_v[r, pl.ds(c * _LANES, _LANES)]
                for rep in range(rows_per_chunk):
                    plsc.addupdate(
                        rows_v.at[rep * seq + r, pl.ds(c * _LANES, _LANES)], p)
            return c2

        lax.fori_loop(0, seq, add_body, 0)
        pltpu.sync_copy(rows_v, out_hbm.at[pl.ds(tok_base + g * chunk, chunk)])
        return carry

    lax.fori_loop(0, n_chunks, chunk_body, 0)


def kernel(token_ids, text_table, pos_table):
    batch, seq = token_ids.shape
    vocab, embed = text_table.shape
    n_flat = batch * seq
    chunk = _SEG * _SEGS_PER_CHUNK

    idx_flat = jnp.reshape(token_ids.astype(jnp.int32), (n_flat // _SEG, _SEG))

    mesh = plsc.VectorSubcoreMesh(core_axis_name="c", subcore_axis_name="s")
    body = functools.partial(_embed_body, seq, embed, n_flat)
    out = pl.kernel(
        body,
        out_type=jax.ShapeDtypeStruct((n_flat, embed), jnp.float32),
        mesh=mesh,
        scratch_types=[
            pltpu.VMEM((_SEGS_PER_CHUNK, _SEG), jnp.int32),
            pltpu.VMEM((chunk, embed), jnp.float32),
            pltpu.VMEM((seq, embed), jnp.float32),
            pltpu.SemaphoreType.DMA,
        ],
        compiler_params=pltpu.CompilerParams(use_tc_tiling_on_sc=False),
        name="sc_embed_lookup",
    )(idx_flat, text_table, pos_table)
    return jnp.reshape(out, (batch, seq, embed))
